# baseline (speedup 1.0000x reference)
import jax
import jax.numpy as jnp
from jax.experimental import pallas as pl

VMIN = -1.0
VMAX = 1.0
NUM_WEIGHTS = 63
INV_DPW = (NUM_WEIGHTS - 1) / (VMAX - VMIN)

ROWS = 8 * 4096
COLS = 1024
BLK_R = 2048
GRID = ROWS // BLK_R


def _lerp(nd):
    nd = jnp.minimum(jnp.maximum(nd, 0.0), float(NUM_WEIGHTS - 1))
    li = nd.astype(jnp.int32)
    return li, nd - li.astype(jnp.float32)


def _tc_body(wlo_ref, whi_ref, x_ref, o_ref):
    tab_lo = jnp.broadcast_to(wlo_ref[...], (BLK_R, 128))
    tab_hi = jnp.broadcast_to(whi_ref[...], (BLK_R, 128))
    for k in range(COLS // 128):
        x = x_ref[:, k * 128:(k + 1) * 128]
        li, f = _lerp(x * INV_DPW + (-VMIN * INV_DPW))
        lo = jnp.take_along_axis(tab_lo, li, axis=-1,
                                 mode="promise_in_bounds")
        hi = jnp.take_along_axis(tab_hi, li, axis=-1,
                                 mode="promise_in_bounds")
        o_ref[:, k * 128:(k + 1) * 128] = lo + f * (hi - lo)


@jax.jit
def kernel(x, weight):
    pad = weight[-1:]
    w_lo = jnp.tile(jnp.concatenate([weight, pad]).reshape(1, 64), (1, 2))
    w_hi = jnp.tile(jnp.concatenate([weight[1:], pad, pad]).reshape(1, 64),
                    (1, 2))
    x2 = x.reshape(ROWS, COLS)
    y = pl.pallas_call(
        _tc_body,
        grid=(GRID,),
        in_specs=[
            pl.BlockSpec((1, 128), lambda i: (0, 0)),
            pl.BlockSpec((1, 128), lambda i: (0, 0)),
            pl.BlockSpec((BLK_R, COLS), lambda i: (i, 0)),
        ],
        out_specs=pl.BlockSpec((BLK_R, COLS), lambda i: (i, 0)),
        out_shape=jax.ShapeDtypeStruct((ROWS, COLS), jnp.float32),
    )(w_lo, w_hi, x2)
    return y.reshape(x.shape)


# speedup vs baseline: 1.1342x; 1.1342x over previous
import functools

import jax
import jax.numpy as jnp
from jax import lax
from jax.experimental import pallas as pl
from jax.experimental.pallas import tpu as pltpu
from jax.experimental.pallas import tpu_sc as plsc

VMIN = -1.0
VMAX = 1.0
NUM_WEIGHTS = 63
INV_DPW = (NUM_WEIGHTS - 1) / (VMAX - VMIN)

ROWS = 8 * 4096
COLS = 1024
R_SC = 16384
NCORES = 2
NSUB = 16
NWORK = NCORES * NSUB
ROWS_W = R_SC // NWORK
CHR = 16
NCHUNK = ROWS_W // CHR
NVEC = CHR * COLS // 16
LANES = 16

BLK_R = 1024
TC_GRID = (ROWS - R_SC) // BLK_R
TC_OFF = R_SC // BLK_R


def _lerp(nd):
    nd = jnp.minimum(jnp.maximum(nd, 0.0), float(NUM_WEIGHTS - 1))
    li = nd.astype(jnp.int32)
    return li, nd - li.astype(jnp.float32)


def _sc_body(x_hbm, w_hbm, out_hbm, lut_lo, lut_hi, in_v, out_v,
             in_sem0, in_sem1, out_sem0, out_sem1):
    wid = lax.axis_index("s") * NCORES + lax.axis_index("c")
    base = wid * ROWS_W

    in_sems = (in_sem0, in_sem1)
    out_sems = (out_sem0, out_sem1)

    pltpu.sync_copy(w_hbm.at[pl.ds(0, 64)], lut_lo)
    pltpu.sync_copy(w_hbm.at[pl.ds(64, 64)], lut_hi)

    def start_in(g, b):
        pltpu.async_copy(x_hbm.at[pl.ds(base + g * CHR, CHR)], in_v.at[b],
                         in_sems[b])

    def wait_in(g, b):
        pltpu.make_async_copy(x_hbm.at[pl.ds(base + g * CHR, CHR)],
                              in_v.at[b], in_sems[b]).wait()

    def start_out(g, b):
        pltpu.async_copy(out_v.at[b], out_hbm.at[pl.ds(base + g * CHR, CHR)],
                         out_sems[b])

    def wait_out(g, b):
        pltpu.make_async_copy(out_v.at[b],
                              out_hbm.at[pl.ds(base + g * CHR, CHR)],
                              out_sems[b]).wait()

    def compute(b):
        @plsc.parallel_loop(0, NVEC, 1, unroll=8)
        def _(i):
            r = lax.shift_right_logical(i, 6)
            c = lax.shift_left(lax.bitwise_and(i, 63), 4)
            x = in_v[b, r, pl.ds(c, LANES)]
            li, f = _lerp(x * INV_DPW + (-VMIN * INV_DPW))
            w_lo = plsc.load_gather(lut_lo, [li])
            w_hi = plsc.load_gather(lut_hi, [li])
            out_v[b, r, pl.ds(c, LANES)] = w_lo + f * (w_hi - w_lo)

    start_in(0, 0)

    def step(i, _):
        for b in (0, 1):
            g = 2 * i + b

            @pl.when(g + 1 < NCHUNK)
            def _():
                start_in(g + 1, (b + 1) % 2)

            wait_in(g, b)

            @pl.when(g >= 2)
            def _():
                wait_out(g - 2, b)

            compute(b)
            start_out(g, b)
        return 0

    lax.fori_loop(0, NCHUNK // 2, step, 0)

    for b in (0, 1):
        wait_out(NCHUNK - 2 + b, b)


_mesh = plsc.VectorSubcoreMesh(core_axis_name="c", subcore_axis_name="s")

_sc_act = functools.partial(
    pl.kernel,
    out_type=jax.ShapeDtypeStruct((R_SC, COLS), jnp.float32),
    mesh=_mesh,
    compiler_params=pltpu.CompilerParams(needs_layout_passes=False,
                                         use_tc_tiling_on_sc=True),
    scratch_types=[
        pltpu.VMEM((64,), jnp.float32),
        pltpu.VMEM((64,), jnp.float32),
        pltpu.VMEM((2, CHR, COLS), jnp.float32),
        pltpu.VMEM((2, CHR, COLS), jnp.float32),
        pltpu.SemaphoreType.DMA,
        pltpu.SemaphoreType.DMA,
        pltpu.SemaphoreType.DMA,
        pltpu.SemaphoreType.DMA,
    ],
)(_sc_body)


def _tc_body(wlo_ref, whi_ref, x_ref, o_ref):
    tab_lo = jnp.broadcast_to(wlo_ref[...], (BLK_R, 128))
    tab_hi = jnp.broadcast_to(whi_ref[...], (BLK_R, 128))
    for k in range(COLS // 128):
        x = x_ref[:, k * 128:(k + 1) * 128]
        li, f = _lerp(x * INV_DPW + (-VMIN * INV_DPW))
        lo = jnp.take_along_axis(tab_lo, li, axis=-1,
                                 mode="promise_in_bounds")
        hi = jnp.take_along_axis(tab_hi, li, axis=-1,
                                 mode="promise_in_bounds")
        o_ref[:, k * 128:(k + 1) * 128] = lo + f * (hi - lo)


@jax.jit
def kernel(x, weight):
    pad = weight[-1:]
    w_lo = jnp.concatenate([weight, pad])
    w_hi = jnp.concatenate([weight[1:], pad, pad])
    x2 = x.reshape(ROWS, COLS)

    y_sc = _sc_act(x2, jnp.concatenate([w_lo, w_hi]))

    wlo2 = jnp.tile(w_lo.reshape(1, 64), (1, 2))
    whi2 = jnp.tile(w_hi.reshape(1, 64), (1, 2))
    y_tc = pl.pallas_call(
        _tc_body,
        grid=(TC_GRID,),
        in_specs=[
            pl.BlockSpec((1, 128), lambda i: (0, 0)),
            pl.BlockSpec((1, 128), lambda i: (0, 0)),
            pl.BlockSpec((BLK_R, COLS), lambda i: (i + TC_OFF, 0)),
        ],
        out_specs=pl.BlockSpec((BLK_R, COLS), lambda i: (i + TC_OFF, 0)),
        out_shape=jax.ShapeDtypeStruct((ROWS, COLS), jnp.float32),
    )(wlo2, whi2, x2)

    y = lax.dynamic_update_slice(y_tc, y_sc, (0, 0))
    return y.reshape(x.shape)
